# baseline (speedup 1.0000x reference)
import functools

import jax
import jax.numpy as jnp
from jax import lax
from jax.experimental import pallas as pl
from jax.experimental.pallas import tpu as pltpu
from jax.experimental.pallas import tpu_sc as plsc

B = 4
S = 2048
S_ENC = 2048
D = 1024
HEADS = 16

NC = 2
NS = 16
NW = NC * NS
TPW = B * S // NW
SPW = S // TPW
NBUF = 4
CH = 16
NCHUNK = TPW // CH

RB = 512
NR = S // RB


@functools.partial(
    pl.kernel,
    out_type=jax.ShapeDtypeStruct((B, S, D), jnp.float32),
    mesh=plsc.VectorSubcoreMesh(core_axis_name="c", subcore_axis_name="s"),
    scratch_types=[pltpu.VMEM((TPW,), jnp.int32)]
                  + [pltpu.VMEM((CH, D), jnp.float32)] * NBUF
                  + [pltpu.SemaphoreType.DMA] * (2 * NBUF),
)
def _embed_gather(table_hbm, ids_hbm, out_hbm, idx_v, *scratch):
    bufs = scratch[:NBUF]
    g_sems = scratch[NBUF:2 * NBUF]
    o_sems = scratch[2 * NBUF:]
    wid = lax.axis_index("s") * NC + lax.axis_index("c")
    b = wid // SPW
    s0 = (wid % SPW) * TPW

    pltpu.sync_copy(ids_hbm.at[b, pl.ds(s0, TPW)], idx_v)

    def gather_start(g):
        cp = pltpu.make_async_copy(
            table_hbm.at[idx_v.at[pl.ds(g * CH, CH)]], bufs[g % NBUF],
            g_sems[g % NBUF])
        cp.start()
        return cp

    def out_start(g):
        cp = pltpu.make_async_copy(
            bufs[g % NBUF], out_hbm.at[b, pl.ds(s0 + g * CH, CH)],
            o_sems[g % NBUF])
        cp.start()
        return cp

    pending_g = {}
    pending_o = {}
    for g in range(min(NBUF - 1, NCHUNK)):
        pending_g[g] = gather_start(g)
    for g in range(NCHUNK):
        pending_g[g].wait()
        pending_o[g] = out_start(g)
        nxt = g + NBUF - 1
        if nxt < NCHUNK:
            if nxt - NBUF >= 0:
                pending_o[nxt - NBUF].wait()
            pending_g[nxt] = gather_start(nxt)
    for g in range(max(0, NCHUNK - NBUF), NCHUNK):
        pending_o[g].wait()


def _mask_body(dec_ref, enc_ref, ext_ref, encext_ref,
               buf0, buf1, sem0, sem1):
    p = pl.program_id(0)
    bufs = (buf0, buf1)
    sems = (sem0, sem1)
    col = lax.broadcasted_iota(jnp.int32, (S, S), 1)
    row = lax.broadcasted_iota(jnp.int32, (S, S), 0)
    causal = col <= row
    for q in range(2):
        b = p * 2 + q
        cp = pltpu.make_async_copy(bufs[q], ext_ref.at[b, 0, :, :], sems[q])

        @pl.when(p > 0)
        def _wait_prev():
            cp.wait()

        m = dec_ref[pl.ds(b, 1), :]
        on_diag = -10000.0 * (1.0 - m)
        bufs[q][...] = jnp.where(causal, on_diag, -10000.0)
        cp.start()

    @pl.when(p == B // 2 - 1)
    def _drain():
        for q in range(2):
            b = p * 2 + q
            pltpu.make_async_copy(bufs[q], ext_ref.at[b, 0, :, :], sems[q]).wait()

    e0 = (1.0 - enc_ref[pl.ds(p * 2, 1), :]) * -1e9
    e1 = (1.0 - enc_ref[pl.ds(p * 2 + 1, 1), :]) * -1e9
    encext_ref[...] = jnp.concatenate([e0, e1], axis=0).reshape(2, 1, 1, S_ENC)


_mask_call = pl.pallas_call(
    _mask_body,
    grid=(B // 2,),
    in_specs=[
        pl.BlockSpec((B, S), lambda p: (0, 0)),
        pl.BlockSpec((B, S_ENC), lambda p: (0, 0)),
    ],
    out_specs=[
        pl.BlockSpec(memory_space=pl.ANY),
        pl.BlockSpec((2, 1, 1, S_ENC), lambda p: (p, 0, 0, 0)),
    ],
    out_shape=[
        jax.ShapeDtypeStruct((B, 1, S, S), jnp.float32),
        jax.ShapeDtypeStruct((B, 1, 1, S_ENC), jnp.float32),
    ],
    scratch_shapes=[pltpu.VMEM((S, S), jnp.float32)] * 2
                   + [pltpu.SemaphoreType.DMA] * 2,
)


def kernel(decoder_input_ids, decoder_attention_mask, encoder_attention_mask, embed_weight):
    hidden = _embed_gather(embed_weight, decoder_input_ids)
    ext, encext = _mask_call(decoder_attention_mask, encoder_attention_mask)
    bias = jnp.zeros((B, HEADS, S, 1), jnp.float32)
    return (hidden, encext, ext, bias)


# speedup vs baseline: 1.0304x; 1.0304x over previous
import functools

import jax
import jax.numpy as jnp
from jax import lax
from jax.experimental import pallas as pl
from jax.experimental.pallas import tpu as pltpu
from jax.experimental.pallas import tpu_sc as plsc

B = 4
S = 2048
S_ENC = 2048
D = 1024
HEADS = 16

NC = 2
NS = 16
NW = NC * NS
TPW = B * S // NW
SPW = S // TPW
CH = 32
NCHUNK = TPW // CH

RB = 512


@functools.partial(
    pl.kernel,
    out_type=jax.ShapeDtypeStruct((B, S, D), jnp.float32),
    mesh=plsc.VectorSubcoreMesh(core_axis_name="c", subcore_axis_name="s"),
    scratch_types=[
        pltpu.VMEM((TPW,), jnp.int32),
        pltpu.VMEM((CH, D), jnp.float32),
        pltpu.VMEM((CH, D), jnp.float32),
        pltpu.SemaphoreType.DMA,
        pltpu.SemaphoreType.DMA,
        pltpu.SemaphoreType.DMA,
        pltpu.SemaphoreType.DMA,
    ],
)
def _embed_gather(table_hbm, ids_hbm, out_hbm, idx_v, rows_a, rows_b,
                  g_sem_a, g_sem_b, o_sem_a, o_sem_b):
    wid = lax.axis_index("s") * NC + lax.axis_index("c")
    b = wid // SPW
    s0 = (wid % SPW) * TPW
    bufs = (rows_a, rows_b)
    g_sems = (g_sem_a, g_sem_b)
    o_sems = (o_sem_a, o_sem_b)

    pltpu.sync_copy(ids_hbm.at[b, pl.ds(s0, TPW)], idx_v)

    def gather_start(g):
        cp = pltpu.make_async_copy(
            table_hbm.at[idx_v.at[pl.ds(g * CH, CH)]], bufs[g % 2], g_sems[g % 2])
        cp.start()
        return cp

    def out_start(g):
        cp = pltpu.make_async_copy(
            bufs[g % 2], out_hbm.at[b, pl.ds(s0 + g * CH, CH)], o_sems[g % 2])
        cp.start()
        return cp

    pending_g = {0: gather_start(0)}
    pending_o = {}
    for g in range(NCHUNK):
        if g + 1 < NCHUNK:
            if g - 1 >= 0:
                pending_o[g - 1].wait()
            pending_g[g + 1] = gather_start(g + 1)
        pending_g[g].wait()
        pending_o[g] = out_start(g)
    pending_o[NCHUNK - 2].wait()
    pending_o[NCHUNK - 1].wait()


def _mask_body(dec_ref, enc_ref, ext_ref, encext_ref):
    b = pl.program_id(0)
    r = pl.program_id(1)
    row = lax.broadcasted_iota(jnp.int32, (RB, S), 0) + r * RB
    col = lax.broadcasted_iota(jnp.int32, (RB, S), 1)
    m = dec_ref[pl.ds(b, 1), :]
    on_diag = -10000.0 * (1.0 - m)
    ext_ref[0, 0] = jnp.where(col <= row, on_diag, -10000.0)
    encext_ref[...] = ((1.0 - enc_ref[pl.ds(b, 1), :]) * -1e9).reshape(1, 1, 1, S_ENC)


_mask_call = pl.pallas_call(
    _mask_body,
    grid=(B, S // RB),
    in_specs=[
        pl.BlockSpec((B, S), lambda b, r: (0, 0)),
        pl.BlockSpec((B, S_ENC), lambda b, r: (0, 0)),
    ],
    out_specs=[
        pl.BlockSpec((1, 1, RB, S), lambda b, r: (b, 0, r, 0)),
        pl.BlockSpec((1, 1, 1, S_ENC), lambda b, r: (b, 0, 0, 0)),
    ],
    out_shape=[
        jax.ShapeDtypeStruct((B, 1, S, S), jnp.float32),
        jax.ShapeDtypeStruct((B, 1, 1, S_ENC), jnp.float32),
    ],
)


def kernel(decoder_input_ids, decoder_attention_mask, encoder_attention_mask, embed_weight):
    hidden = _embed_gather(embed_weight, decoder_input_ids)
    ext, encext = _mask_call(decoder_attention_mask, encoder_attention_mask)
    bias = jnp.zeros((B, HEADS, S, 1), jnp.float32)
    return (hidden, encext, ext, bias)
